# baseline (speedup 1.0000x reference)
import jax
import jax.numpy as jnp
from jax import lax
from jax.experimental import pallas as pl
from jax.experimental.pallas import tpu as pltpu
from jax.experimental.pallas import tpu_sc as plsc

_SEQ_LEN = 16384
_VOCAB = 1000
_PAD = 0

_NC = 2
_NS = 16
_L = 16
_NW = _NC * _NS
_RPW = _SEQ_LEN // _NW
_NIDX = 128
_NXFER = _RPW // _NIDX

_ZBLK = 2_048_000
_TOTAL = _SEQ_LEN * _VOCAB


def _zero_body(o_ref):
    o_ref[...] = jnp.zeros((_ZBLK,), jnp.float32)


def _tc_zeros():
    return pl.pallas_call(
        _zero_body,
        out_shape=jax.ShapeDtypeStruct((_TOTAL,), jnp.float32),
        grid=(_TOTAL // _ZBLK,),
        out_specs=pl.BlockSpec((_ZBLK,), lambda i: (i,)),
    )()


def _scatter_body(seq_hbm, vals_hbm, out_ref, seq_v, vals_v, idx_v, src_v,
                  sem):
    wid = lax.axis_index("s") * _NC + lax.axis_index("c")
    base = wid * _RPW

    pltpu.sync_copy(seq_hbm.at[pl.ds(base, _RPW)], seq_v)
    pltpu.sync_copy(vals_hbm.at[pl.ds(base, _RPW)], vals_v)

    lane = lax.iota(jnp.int32, _L)

    for t in range(_RPW // _L):
        seq16 = seq_v[pl.ds(t * _L, _L)]
        v16 = vals_v[pl.ds(t * _L, _L)]
        gidx = (base + t * _L + lane) * _VOCAB + seq16
        val = jnp.where(seq16 != _PAD, v16, jnp.zeros((_L,), jnp.float32))
        j, c0 = divmod(t * _L, _NIDX)
        idx_v[j, pl.ds(c0, _L)] = gidx
        src_v[j, pl.ds(c0, _L)] = val

    copies = [
        pltpu.async_copy(src_v.at[j], out_ref.at[idx_v.at[j]], sem)
        for j in range(_NXFER)
    ]
    for cp in copies:
        cp.wait()


def _sc_scatter(sequence, vals, out_ref):
    mesh = plsc.VectorSubcoreMesh(core_axis_name="c", subcore_axis_name="s")
    pl.kernel(
        _scatter_body,
        mesh=mesh,
        compiler_params=pltpu.CompilerParams(needs_layout_passes=False),
        out_type=(),
        scratch_types=[
            pltpu.VMEM((_RPW,), jnp.int32),
            pltpu.VMEM((_RPW,), jnp.float32),
            pltpu.VMEM((_NXFER, _NIDX), jnp.int32),
            pltpu.VMEM((_NXFER, _NIDX), jnp.float32),
            pltpu.SemaphoreType.DMA,
        ],
    )(sequence, vals, out_ref)


@jax.jit
def kernel(sequence, vals):
    buf = jax.new_ref(_tc_zeros())
    _sc_scatter(sequence, vals, buf)
    return jax.ref.freeze(buf).reshape(_SEQ_LEN, _VOCAB)
